# baseline (speedup 1.0000x reference)
import jax
import jax.numpy as jnp
from jax import lax
from jax.experimental import pallas as pl
from jax.experimental.pallas import tpu as pltpu

_N_CLS = 1000
_ROWS = 16384
_SCALE = 1.0 / (_ROWS * _N_CLS)

_BJ = 8
_NSJ = 25
_NSB = 2
_BB = _ROWS // _NSB
_GRID = _N_CLS // (_BJ * _NSJ)


def _tc_body(t_ref, *rest):
    p_refs, o_ref = rest[:_NSJ * _NSB], rest[_NSJ * _NSB]
    i = pl.program_id(0)
    t = t_ref[...]
    partial = jnp.float32(0.0)
    for s in range(_NSJ):
        base = (_NSJ * i + s) * _BJ
        j = lax.broadcasted_iota(jnp.int32, (_BJ, _BB), 0) + base
        jf = j.astype(jnp.float32)
        for s2 in range(_NSB):
            p_ref = p_refs[s * _NSB + s2]
            tt = t[:, s2 * _BB:(s2 + 1) * _BB]
            w = jnp.abs(jf - tt) * jnp.float32(_SCALE)
            partial += jnp.sum(w * p_ref[...])

    @pl.when(i == 0)
    def _init():
        o_ref[0, 0] = 0.0

    o_ref[0, 0] += partial


def kernel(output_probs, target_class):
    pt = output_probs.T
    t_row = target_class.astype(jnp.float32).reshape(1, _ROWS)
    in_specs = [pl.BlockSpec((1, _ROWS), lambda i: (0, 0))]
    for s in range(_NSJ):
        for s2 in range(_NSB):
            in_specs.append(
                pl.BlockSpec((_BJ, _BB),
                             lambda i, s=s, s2=s2: (_NSJ * i + s, s2)))
    out = pl.pallas_call(
        _tc_body,
        grid=(_GRID,),
        in_specs=in_specs,
        out_specs=pl.BlockSpec(memory_space=pltpu.SMEM),
        out_shape=jax.ShapeDtypeStruct((1, 1), jnp.float32),
    )(t_row, *([pt] * (_NSJ * _NSB)))
    return out[0, 0]


# speedup vs baseline: 1.0412x; 1.0412x over previous
import jax
import jax.numpy as jnp
from jax import lax
from jax.experimental import pallas as pl
from jax.experimental.pallas import tpu as pltpu

_N_CLS = 1000
_ROWS = 16384
_SCALE = 1.0 / (_ROWS * _N_CLS)

_BJ = 8
_NSTREAM = 25
_GRID = _N_CLS // (_BJ * _NSTREAM)


def _tc_body(t_ref, *rest):
    p_refs, o_ref = rest[:_NSTREAM], rest[_NSTREAM]
    i = pl.program_id(0)
    t = t_ref[...]
    partial = jnp.float32(0.0)
    for s, p_ref in enumerate(p_refs):
        base = (_GRID * s + i) * _BJ
        j = lax.broadcasted_iota(jnp.int32, (_BJ, _ROWS), 0) + base
        w = jnp.abs(j.astype(jnp.float32) - t) * jnp.float32(_SCALE)
        partial += jnp.sum(w * p_ref[...])

    @pl.when(i == 0)
    def _init():
        o_ref[0, 0] = 0.0

    o_ref[0, 0] += partial


def kernel(output_probs, target_class):
    pt = output_probs.T
    t_row = target_class.astype(jnp.float32).reshape(1, _ROWS)
    in_specs = [pl.BlockSpec((1, _ROWS), lambda i: (0, 0))]
    for s in range(_NSTREAM):
        in_specs.append(
            pl.BlockSpec((_BJ, _ROWS), lambda i, s=s: (_GRID * s + i, 0)))
    out = pl.pallas_call(
        _tc_body,
        grid=(_GRID,),
        in_specs=in_specs,
        out_specs=pl.BlockSpec(memory_space=pltpu.SMEM),
        out_shape=jax.ShapeDtypeStruct((1, 1), jnp.float32),
    )(t_row, *([pt] * _NSTREAM))
    return out[0, 0]
